# baseline (speedup 1.0000x reference)
import functools

import jax
import jax.numpy as jnp
from jax import lax
from jax.experimental import pallas as pl
from jax.experimental.pallas import tpu as pltpu
from jax.experimental.pallas import tpu_sc as plsc

N_VOCAB = 4 * 10 ** 6
DIM = 5
LANES = 128
T_ROWS = N_VOCAB // LANES
BLKR = 2048
BLKL = BLKR * LANES

B_TOTAL = 16384 * 200
IDX_COLS = 128
NC, NS = 2, 16
NW = NC * NS
K = 16
CHUNK = K * IDX_COLS
N_CHUNKS = B_TOTAL // (NW * CHUNK)
SUBS = 16384 // CHUNK


def _fuse_body(embT_ref, wcol_ref, b_ref, out_ref):
    x = embT_ref[...]
    xr = x.reshape(DIM * BLKR, LANES)
    acc = xr[0:BLKR] * wcol_ref[0, 0]
    for k in range(1, DIM):
        acc = acc + xr[k * BLKR:(k + 1) * BLKR] * wcol_ref[k, 0]
    out_ref[...] = acc + b_ref[0]


def _fuse_table(embT, W, b):
    grid = (N_VOCAB + BLKL - 1) // BLKL
    return pl.pallas_call(
        _fuse_body,
        grid=(grid,),
        in_specs=[
            pl.BlockSpec((DIM, BLKL), lambda i: (0, i)),
            pl.BlockSpec(memory_space=pltpu.SMEM),
            pl.BlockSpec(memory_space=pltpu.SMEM),
        ],
        out_specs=pl.BlockSpec((BLKR, LANES), lambda i: (i, 0)),
        out_shape=jax.ShapeDtypeStruct((T_ROWS, LANES), jnp.float32),
    )(embT, W.T, b)


def _gather_body(t_hbm, idx_hbm, out_hbm, idx_v0, idx_v1, idx_v2,
                 val_v0, val_v1, val_v2,
                 isem0, isem1, isem2, gsem0, gsem1, gsem2,
                 osem0, osem1, osem2):
    idx_v = (idx_v0, idx_v1, idx_v2)
    val_v = (val_v0, val_v1, val_v2)
    isem = (isem0, isem1, isem2)
    gsem = (gsem0, gsem1, gsem2)
    osem = (osem0, osem1, osem2)
    wid = lax.axis_index("s") * NC + lax.axis_index("c")
    base = wid * N_CHUNKS

    def start_idx(c, b):
        g = base + c
        pltpu.async_copy(
            idx_hbm.at[g // SUBS].at[pl.ds((g % SUBS) * CHUNK, CHUNK)],
            idx_v[b], isem[b])

    def wait_idx(b):
        pltpu.make_async_copy(idx_hbm.at[0].at[pl.ds(0, CHUNK)],
                              idx_v[b], isem[b]).wait()

    def fire(c, b):
        del c
        for j in range(K):
            sl = pl.ds(j * IDX_COLS, IDX_COLS)
            pltpu.async_copy(t_hbm.at[idx_v[b].at[sl]],
                             val_v[b].at[sl], gsem[b])

    def drain(b):
        pltpu.make_async_copy(out_hbm.at[0].at[pl.ds(0, CHUNK)], val_v[b],
                              gsem[b]).wait()

    def start_out(c, b):
        g = base + c
        pltpu.async_copy(val_v[b],
                         out_hbm.at[g // SUBS].at[pl.ds((g % SUBS) * CHUNK,
                                                        CHUNK)],
                         osem[b])

    def wait_out(b):
        pltpu.make_async_copy(val_v[b], out_hbm.at[0].at[pl.ds(0, CHUNK)],
                              osem[b]).wait()

    def step(c, b):
        b2 = (b + 2) % 3
        wait_out(b2)
        wait_idx(b2)
        fire(c + 2, b2)
        drain(b)
        start_out(c, b)
        start_idx(c + 3, b)

    start_idx(0, 0)
    start_idx(1, 1)
    start_idx(2, 2)
    wait_idx(0)
    fire(0, 0)
    wait_idx(1)
    fire(1, 1)
    wait_idx(2)
    fire(2, 2)
    drain(0)
    start_out(0, 0)
    start_idx(3, 0)

    def triple(g, carry):
        step(3 * g + 1, 1)
        step(3 * g + 2, 2)
        step(3 * g + 3, 0)
        return carry

    lax.fori_loop(0, (N_CHUNKS - 5) // 3, triple, 0)

    c = N_CHUNKS - 4
    step(c, 1)
    c = N_CHUNKS - 3
    wait_out(1)
    wait_idx(1)
    fire(c + 2, 1)
    drain(2)
    start_out(c, 2)
    c = N_CHUNKS - 2
    wait_out(2)
    drain(0)
    start_out(c, 0)
    c = N_CHUNKS - 1
    drain(1)
    start_out(c, 1)
    wait_out(0)
    wait_out(1)


@functools.cache
def _gather_kernel():
    return pl.kernel(
        _gather_body,
        mesh=plsc.VectorSubcoreMesh(core_axis_name="c", subcore_axis_name="s"),
        out_type=jax.ShapeDtypeStruct((200, 16384), jnp.float32),
        scratch_types=(
            [pltpu.VMEM((CHUNK,), jnp.int32)] * 3
            + [pltpu.VMEM((CHUNK,), jnp.float32)] * 3
            + [pltpu.SemaphoreType.DMA] * 9
        ),
    )


def kernel(input, emb, W, b):
    t = _fuse_table(emb.T, W, b)
    idx = input.T.astype(jnp.int32)
    out = _gather_kernel()(t.reshape(N_VOCAB), idx)
    return out.T.reshape(16384, 200, 1)
